# baseline (speedup 1.0000x reference)
import functools

import jax
import jax.numpy as jnp
from jax import lax
from jax.experimental import pallas as pl
from jax.experimental.pallas import tpu as pltpu
from jax.experimental.pallas import tpu_sc as plsc


@functools.cache
def _sc_fanout(n, t, d, dtype_name):
    dtype = jnp.dtype(dtype_name)
    info = plsc.get_sparse_core_info()
    nc, ns = info.num_cores, info.num_subcores
    nw = nc * ns
    rows_per_w = t // nw
    mesh = plsc.VectorSubcoreMesh(core_axis_name="c", subcore_axis_name="s")

    nchunk = 4
    rows_per_c = rows_per_w // nchunk

    @functools.partial(
        pl.kernel,
        mesh=mesh,
        out_type=jax.ShapeDtypeStruct((n, t, d), dtype),
        scratch_types=[
            pltpu.VMEM((rows_per_w, d), dtype),
            pltpu.SemaphoreType.DMA((nchunk,)),
            pltpu.SemaphoreType.DMA,
        ],
    )
    def k(table_hbm, out_hbm, rows_v, gsems, ssem):
        wid = lax.axis_index("s") * nc + lax.axis_index("c")
        base = wid * rows_per_w
        gathers = [
            pltpu.async_copy(
                table_hbm.at[pl.ds(base + j * rows_per_c, rows_per_c)],
                rows_v.at[pl.ds(j * rows_per_c, rows_per_c)],
                gsems.at[j],
            )
            for j in range(nchunk)
        ]
        scatters = []
        for j in range(nchunk):
            gathers[j].wait()
            scatters.extend(
                pltpu.async_copy(
                    rows_v.at[pl.ds(j * rows_per_c, rows_per_c)],
                    out_hbm.at[i, pl.ds(base + j * rows_per_c, rows_per_c)],
                    ssem,
                )
                for i in range(n)
            )
        for c in scatters:
            c.wait()

    return k


def _tc_body(p_ref, o_ref):
    o_ref[...] = jnp.broadcast_to(p_ref[...][None], o_ref.shape)


def _tc_fanout(n, t, d, dtype, bt=512):
    return pl.pallas_call(
        _tc_body,
        grid=(t // bt,),
        in_specs=[pl.BlockSpec((bt, d), lambda i: (i, 0))],
        out_specs=pl.BlockSpec((n, bt, d), lambda i: (0, i, 0)),
        out_shape=jax.ShapeDtypeStruct((n, t, d), dtype),
    )


def kernel(inputs, params):
    n, t, d = inputs.shape
    n_sc = 1
    sc_out = _sc_fanout(n_sc, t, d, str(params.dtype))(params)
    tc_out = _tc_fanout(n - n_sc, t, d, params.dtype)(params)
    return jnp.concatenate([tc_out, sc_out], axis=0)


# speedup vs baseline: 1.6982x; 1.6982x over previous
import functools

import jax
import jax.numpy as jnp
from jax import lax
from jax.experimental import pallas as pl
from jax.experimental.pallas import tpu as pltpu
from jax.experimental.pallas import tpu_sc as plsc


@functools.cache
def _sc_fanout(n, t, d, dtype_name):
    dtype = jnp.dtype(dtype_name)
    info = plsc.get_sparse_core_info()
    nc, ns = info.num_cores, info.num_subcores
    nw = nc * ns
    rows_per_w = t // nw
    mesh = plsc.VectorSubcoreMesh(core_axis_name="c", subcore_axis_name="s")

    nchunk = 4
    rows_per_c = rows_per_w // nchunk

    @functools.partial(
        pl.kernel,
        mesh=mesh,
        out_type=jax.ShapeDtypeStruct((n, t, d), dtype),
        scratch_types=[
            pltpu.VMEM((rows_per_w, d), dtype),
            pltpu.SemaphoreType.DMA((nchunk,)),
            pltpu.SemaphoreType.DMA,
        ],
    )
    def k(table_hbm, out_hbm, rows_v, gsems, ssem):
        wid = lax.axis_index("s") * nc + lax.axis_index("c")
        base = wid * rows_per_w
        gathers = [
            pltpu.async_copy(
                table_hbm.at[pl.ds(base + j * rows_per_c, rows_per_c)],
                rows_v.at[pl.ds(j * rows_per_c, rows_per_c)],
                gsems.at[j],
            )
            for j in range(nchunk)
        ]
        scatters = []
        for j in range(nchunk):
            gathers[j].wait()
            scatters.extend(
                pltpu.async_copy(
                    rows_v.at[pl.ds(j * rows_per_c, rows_per_c)],
                    out_hbm.at[i, pl.ds(base + j * rows_per_c, rows_per_c)],
                    ssem,
                )
                for i in range(n)
            )
        for c in scatters:
            c.wait()

    return k


def _tc_body(p_ref, o_ref):
    o_ref[...] = jnp.broadcast_to(p_ref[...][None], o_ref.shape)


def _tc_fanout(n, t, d, dtype, bt=512):
    return pl.pallas_call(
        _tc_body,
        grid=(t // bt,),
        in_specs=[pl.BlockSpec((bt, d), lambda i: (i, 0))],
        out_specs=pl.BlockSpec((n, bt, d), lambda i: (0, i, 0)),
        out_shape=jax.ShapeDtypeStruct((n, t, d), dtype),
    )


def kernel(inputs, params):
    n, t, d = inputs.shape
    return _sc_fanout(n, t, d, str(params.dtype))(params)


# speedup vs baseline: 1.6992x; 1.0006x over previous
import functools

import jax
import jax.numpy as jnp
from jax import lax
from jax.experimental import pallas as pl
from jax.experimental.pallas import tpu as pltpu
from jax.experimental.pallas import tpu_sc as plsc


@functools.cache
def _sc_fanout(n, t, d, dtype_name):
    dtype = jnp.dtype(dtype_name)
    info = plsc.get_sparse_core_info()
    nc, ns = info.num_cores, info.num_subcores
    nw = nc * ns
    rows_per_w = t // nw
    mesh = plsc.VectorSubcoreMesh(core_axis_name="c", subcore_axis_name="s")

    nchunk = 4
    rows_per_c = rows_per_w // nchunk

    @functools.partial(
        pl.kernel,
        mesh=mesh,
        out_type=jax.ShapeDtypeStruct((n, t, d), dtype),
        scratch_types=[
            pltpu.VMEM((rows_per_w, d), dtype),
            pltpu.SemaphoreType.DMA((nchunk,)),
            pltpu.SemaphoreType.DMA,
        ],
    )
    def k(table_hbm, out_hbm, rows_v, gsems, ssem):
        wid = lax.axis_index("s") * nc + lax.axis_index("c")
        base = wid * rows_per_w
        gathers = [
            pltpu.async_copy(
                table_hbm.at[pl.ds(base + j * rows_per_c, rows_per_c)],
                rows_v.at[pl.ds(j * rows_per_c, rows_per_c)],
                gsems.at[j],
            )
            for j in range(nchunk)
        ]
        scatters = []
        for j in range(nchunk):
            gathers[j].wait()
            for i in range(n):
                slot = lax.rem(wid + i + j, n)
                scatters.append(
                    pltpu.async_copy(
                        rows_v.at[pl.ds(j * rows_per_c, rows_per_c)],
                        out_hbm.at[slot, pl.ds(base + j * rows_per_c, rows_per_c)],
                        ssem,
                    )
                )
        for c in scatters:
            c.wait()

    return k


def _tc_body(p_ref, o_ref):
    o_ref[...] = jnp.broadcast_to(p_ref[...][None], o_ref.shape)


def _tc_fanout(n, t, d, dtype, bt=512):
    return pl.pallas_call(
        _tc_body,
        grid=(t // bt,),
        in_specs=[pl.BlockSpec((bt, d), lambda i: (i, 0))],
        out_specs=pl.BlockSpec((n, bt, d), lambda i: (0, i, 0)),
        out_shape=jax.ShapeDtypeStruct((n, t, d), dtype),
    )


def kernel(inputs, params):
    n, t, d = inputs.shape
    return _sc_fanout(n, t, d, str(params.dtype))(params)


# speedup vs baseline: 1.7242x; 1.0148x over previous
import functools

import jax
import jax.numpy as jnp
from jax import lax
from jax.experimental import pallas as pl
from jax.experimental.pallas import tpu as pltpu
from jax.experimental.pallas import tpu_sc as plsc


@functools.cache
def _sc_fanout(n, t, d, dtype_name):
    dtype = jnp.dtype(dtype_name)
    info = plsc.get_sparse_core_info()
    nc, ns = info.num_cores, info.num_subcores
    nw = nc * ns
    rows_per_w = t // nw
    mesh = plsc.VectorSubcoreMesh(core_axis_name="c", subcore_axis_name="s")

    @functools.partial(
        pl.kernel,
        mesh=mesh,
        out_type=jax.ShapeDtypeStruct((n, t, d), dtype),
        scratch_types=[
            pltpu.VMEM((rows_per_w, d), dtype),
            pltpu.SemaphoreType.DMA,
        ],
    )
    def k(table_hbm, out_hbm, rows_v, sem):
        wid = lax.axis_index("s") * nc + lax.axis_index("c")
        base = wid * rows_per_w
        pltpu.sync_copy(table_hbm.at[pl.ds(base, rows_per_w)], rows_v)
        copies = [
            pltpu.async_copy(rows_v, out_hbm.at[i, pl.ds(base, rows_per_w)], sem)
            for i in range(n)
        ]
        for c in copies:
            c.wait()

    return k


def kernel(inputs, params):
    n, t, d = inputs.shape
    return _sc_fanout(n, t, d, str(params.dtype))(params)
